# Initial kernel scaffold; baseline (speedup 1.0000x reference)
import jax
import jax.numpy as jnp
from jax.experimental import pallas as pl


def kernel(x, edge_index, W1, b1, W2, b2, a):
    raise NotImplementedError("write your pallas kernel here")


# speedup vs baseline: 8.0995x; 8.0995x over previous
import functools

import jax
import jax.numpy as jnp
from jax import lax
from jax.experimental import pallas as pl
from jax.experimental.pallas import tpu as pltpu
from jax.experimental.pallas import tpu_sc as plsc

N = 10000
E = 320000
D_IN = 128
D_HID = 64

NC = 2
NS = 16
NW = NC * NS
E_PER_W = E // NW
CHUNK = 80
NCHUNK = E_PER_W // CHUNK
ROWS_PER_TILE = 624
ROWS_TAIL = N - ROWS_PER_TILE * NS
DEG_W = 16


def _mm1_body(x_ref, w_ref, b_ref, o_ref):
    o_ref[...] = (
        jnp.dot(x_ref[...], w_ref[...], preferred_element_type=jnp.float32)
        + b_ref[...]
    )


def _stage1(x, W1, b1):
    B = 1000
    return pl.pallas_call(
        _mm1_body,
        grid=(N // B,),
        in_specs=[
            pl.BlockSpec((B, D_IN), lambda i: (i, 0)),
            pl.BlockSpec((D_IN, D_HID), lambda i: (0, 0)),
            pl.BlockSpec((1, D_HID), lambda i: (0, 0)),
        ],
        out_specs=pl.BlockSpec((B, D_HID), lambda i: (i, 0)),
        out_shape=jax.ShapeDtypeStruct((N, D_HID), jnp.float32),
    )(x, W1, b1.reshape(1, D_HID))


def _sc_agg_body(h_hbm, src_hbm, dst_hbm, z64_hbm, z16_hbm, ones_hbm,
                 agg_out, deg_out,
                 agg_sh, deg_sh, src_v, dst_v, rows_v, ones_v, sem):
    cid = lax.axis_index("c")
    sid = lax.axis_index("s")
    wid = sid * NC + cid

    rbase = sid * ROWS_PER_TILE
    pltpu.sync_copy(z64_hbm.at[pl.ds(rbase, ROWS_PER_TILE)],
                    agg_sh.at[pl.ds(rbase, ROWS_PER_TILE)])
    pltpu.sync_copy(z16_hbm.at[pl.ds(rbase, ROWS_PER_TILE)],
                    deg_sh.at[pl.ds(rbase, ROWS_PER_TILE)])

    @pl.when(sid == NS - 1)
    def _zero_tail():
        tb = ROWS_PER_TILE * NS
        pltpu.sync_copy(z64_hbm.at[pl.ds(tb, ROWS_TAIL)],
                        agg_sh.at[pl.ds(tb, ROWS_TAIL)])
        pltpu.sync_copy(z16_hbm.at[pl.ds(tb, ROWS_TAIL)],
                        deg_sh.at[pl.ds(tb, ROWS_TAIL)])

    pltpu.sync_copy(src_hbm.at[wid], src_v)
    pltpu.sync_copy(dst_hbm.at[wid], dst_v)
    pltpu.sync_copy(ones_hbm, ones_v)
    plsc.subcore_barrier()

    def body(j, carry):
        pltpu.async_copy(h_hbm.at[src_v.at[j]], rows_v, sem).wait()
        pltpu.sync_copy(rows_v, agg_sh.at[dst_v.at[j]], add=True)
        pltpu.sync_copy(ones_v, deg_sh.at[dst_v.at[j]], add=True)
        return carry

    lax.fori_loop(0, NCHUNK, body, 0)
    plsc.subcore_barrier()

    pltpu.sync_copy(agg_sh.at[pl.ds(rbase, ROWS_PER_TILE)],
                    agg_out.at[cid, pl.ds(rbase, ROWS_PER_TILE)])
    pltpu.sync_copy(deg_sh.at[pl.ds(rbase, ROWS_PER_TILE)],
                    deg_out.at[cid, pl.ds(rbase, ROWS_PER_TILE)])

    @pl.when(sid == NS - 1)
    def _publish_tail():
        tb = ROWS_PER_TILE * NS
        pltpu.sync_copy(agg_sh.at[pl.ds(tb, ROWS_TAIL)],
                        agg_out.at[cid, pl.ds(tb, ROWS_TAIL)])
        pltpu.sync_copy(deg_sh.at[pl.ds(tb, ROWS_TAIL)],
                        deg_out.at[cid, pl.ds(tb, ROWS_TAIL)])


def _stage2(h, src, dst):
    src3 = src.reshape(NW, NCHUNK, CHUNK)
    dst3 = dst.reshape(NW, NCHUNK, CHUNK)
    z64 = jnp.zeros((N, D_HID), jnp.float32)
    z16 = jnp.zeros((N, DEG_W), jnp.float32)
    ones = jnp.ones((CHUNK, DEG_W), jnp.float32)
    mesh = plsc.VectorSubcoreMesh(core_axis_name="c", subcore_axis_name="s")
    f = functools.partial(
        pl.kernel,
        out_type=[
            jax.ShapeDtypeStruct((NC, N, D_HID), jnp.float32),
            jax.ShapeDtypeStruct((NC, N, DEG_W), jnp.float32),
        ],
        mesh=mesh,
        compiler_params=pltpu.CompilerParams(use_tc_tiling_on_sc=False),
        scratch_types=[
            pltpu.VMEM_SHARED((N, D_HID), jnp.float32),
            pltpu.VMEM_SHARED((N, DEG_W), jnp.float32),
            pltpu.VMEM((NCHUNK, CHUNK), jnp.int32),
            pltpu.VMEM((NCHUNK, CHUNK), jnp.int32),
            pltpu.VMEM((CHUNK, D_HID), jnp.float32),
            pltpu.VMEM((CHUNK, DEG_W), jnp.float32),
            pltpu.SemaphoreType.DMA,
        ],
    )(_sc_agg_body)
    return f(h, src3, dst3, z64, z16, ones)


def _fin_body(agg_ref, deg_ref, w_ref, b_ref, a_ref, o_ref):
    s = agg_ref[0] + agg_ref[1]
    d = deg_ref[0, :, 0:1] + deg_ref[1, :, 0:1]
    m = s / jnp.maximum(d, 1.0)
    p = jnp.where(m >= 0, m, a_ref[...] * m)
    o_ref[...] = (
        jnp.dot(p, w_ref[...], preferred_element_type=jnp.float32) + b_ref[...]
    )


def _stage3(aggp, degp, W2, b2, a):
    B = 1000
    a_row = jnp.full((1, D_HID), a, jnp.float32)
    return pl.pallas_call(
        _fin_body,
        grid=(N // B,),
        in_specs=[
            pl.BlockSpec((NC, B, D_HID), lambda i: (0, i, 0)),
            pl.BlockSpec((NC, B, DEG_W), lambda i: (0, i, 0)),
            pl.BlockSpec((D_HID, D_HID), lambda i: (0, 0)),
            pl.BlockSpec((1, D_HID), lambda i: (0, 0)),
            pl.BlockSpec((1, D_HID), lambda i: (0, 0)),
        ],
        out_specs=pl.BlockSpec((B, D_HID), lambda i: (i, 0)),
        out_shape=jax.ShapeDtypeStruct((N, D_HID), jnp.float32),
    )(aggp, degp, W2, b2.reshape(1, D_HID), a_row)


def kernel(x, edge_index, W1, b1, W2, b2, a):
    h = _stage1(x, W1, b1)
    aggp, degp = _stage2(h, edge_index[0], edge_index[1])
    return _stage3(aggp, degp, W2, b2, a)


# speedup vs baseline: 11.7845x; 1.4550x over previous
import functools

import jax
import jax.numpy as jnp
from jax import lax
from jax.experimental import pallas as pl
from jax.experimental.pallas import tpu as pltpu
from jax.experimental.pallas import tpu_sc as plsc

N = 10000
E = 320000
D_IN = 128
D_HID = 64

NC = 2
NS = 16
NW = NC * NS
E_PER_W = E // NW
CHUNK = 100
NCHUNK = E_PER_W // CHUNK
NCHUNK2 = NCHUNK // 2
ROWS_PER_TILE = 624
ROWS_TAIL = N - ROWS_PER_TILE * NS
DEG_W = 16


def _mm1_body(x_ref, w_ref, b_ref, o_ref):
    o_ref[...] = (
        jnp.dot(x_ref[...], w_ref[...], preferred_element_type=jnp.float32)
        + b_ref[...]
    )


def _stage1(x, W1, b1):
    B = 1000
    return pl.pallas_call(
        _mm1_body,
        grid=(N // B,),
        in_specs=[
            pl.BlockSpec((B, D_IN), lambda i: (i, 0)),
            pl.BlockSpec((D_IN, D_HID), lambda i: (0, 0)),
            pl.BlockSpec((1, D_HID), lambda i: (0, 0)),
        ],
        out_specs=pl.BlockSpec((B, D_HID), lambda i: (i, 0)),
        out_shape=jax.ShapeDtypeStruct((N, D_HID), jnp.float32),
    )(x, W1, b1.reshape(1, D_HID))


def _sc_agg_body(h_hbm, src_hbm, dst_hbm, z64_hbm, z16_hbm, ones_hbm,
                 agg_out, deg_out,
                 agg_sh, deg_sh, src_v, dst_v, rows0, rows1, ones_v,
                 sem_g0, sem_g1, sem_s0, sem_s1):
    cid = lax.axis_index("c")
    sid = lax.axis_index("s")
    wid = sid * NC + cid

    rbase = sid * ROWS_PER_TILE
    pltpu.sync_copy(z64_hbm.at[pl.ds(rbase, ROWS_PER_TILE)],
                    agg_sh.at[pl.ds(rbase, ROWS_PER_TILE)])
    pltpu.sync_copy(z16_hbm.at[pl.ds(rbase, ROWS_PER_TILE)],
                    deg_sh.at[pl.ds(rbase, ROWS_PER_TILE)])

    @pl.when(sid == NS - 1)
    def _zero_tail():
        tb = ROWS_PER_TILE * NS
        pltpu.sync_copy(z64_hbm.at[pl.ds(tb, ROWS_TAIL)],
                        agg_sh.at[pl.ds(tb, ROWS_TAIL)])
        pltpu.sync_copy(z16_hbm.at[pl.ds(tb, ROWS_TAIL)],
                        deg_sh.at[pl.ds(tb, ROWS_TAIL)])

    pltpu.sync_copy(src_hbm.at[wid], src_v)
    pltpu.sync_copy(dst_hbm.at[wid], dst_v)
    pltpu.sync_copy(ones_hbm, ones_v)
    plsc.subcore_barrier()

    def _fire_gather(j, buf, sem):
        pltpu.async_copy(h_hbm.at[src_v.at[j]], buf, sem)

    def _wait_gather(j, buf, sem):
        pltpu.make_async_copy(h_hbm.at[src_v.at[j]], buf, sem).wait()

    def _fire_scatter(j, buf, sem):
        pltpu.async_copy(buf, agg_sh.at[dst_v.at[j]], sem, add=True)
        pltpu.async_copy(ones_v, deg_sh.at[dst_v.at[j]], sem, add=True)

    def _wait_scatter(j, buf, sem):
        pltpu.make_async_copy(buf, agg_sh.at[dst_v.at[j]], sem).wait()
        pltpu.make_async_copy(ones_v, deg_sh.at[dst_v.at[j]], sem).wait()

    _fire_gather(0, rows0, sem_g0)

    def body(jj, carry):
        j0 = 2 * jj
        j1 = j0 + 1

        @pl.when(jj > 0)
        def _():
            _wait_scatter(j1, rows1, sem_s1)

        _fire_gather(j1, rows1, sem_g1)
        _wait_gather(j0, rows0, sem_g0)
        _fire_scatter(j0, rows0, sem_s0)
        _wait_scatter(j0, rows0, sem_s0)

        @pl.when(jj + 1 < NCHUNK2)
        def _():
            _fire_gather(j0 + 2, rows0, sem_g0)

        _wait_gather(j1, rows1, sem_g1)
        _fire_scatter(j1, rows1, sem_s1)
        return carry

    lax.fori_loop(0, NCHUNK2, body, 0)
    _wait_scatter(NCHUNK - 1, rows1, sem_s1)
    plsc.subcore_barrier()

    pltpu.sync_copy(agg_sh.at[pl.ds(rbase, ROWS_PER_TILE)],
                    agg_out.at[cid, pl.ds(rbase, ROWS_PER_TILE)])
    pltpu.sync_copy(deg_sh.at[pl.ds(rbase, ROWS_PER_TILE)],
                    deg_out.at[cid, pl.ds(rbase, ROWS_PER_TILE)])

    @pl.when(sid == NS - 1)
    def _publish_tail():
        tb = ROWS_PER_TILE * NS
        pltpu.sync_copy(agg_sh.at[pl.ds(tb, ROWS_TAIL)],
                        agg_out.at[cid, pl.ds(tb, ROWS_TAIL)])
        pltpu.sync_copy(deg_sh.at[pl.ds(tb, ROWS_TAIL)],
                        deg_out.at[cid, pl.ds(tb, ROWS_TAIL)])


def _stage2(h, src, dst):
    src3 = src.reshape(NW, NCHUNK, CHUNK)
    dst3 = dst.reshape(NW, NCHUNK, CHUNK)
    z64 = jnp.zeros((N, D_HID), jnp.float32)
    z16 = jnp.zeros((N, DEG_W), jnp.float32)
    ones = jnp.ones((CHUNK, DEG_W), jnp.float32)
    mesh = plsc.VectorSubcoreMesh(core_axis_name="c", subcore_axis_name="s")
    f = functools.partial(
        pl.kernel,
        out_type=[
            jax.ShapeDtypeStruct((NC, N, D_HID), jnp.float32),
            jax.ShapeDtypeStruct((NC, N, DEG_W), jnp.float32),
        ],
        mesh=mesh,
        compiler_params=pltpu.CompilerParams(use_tc_tiling_on_sc=False),
        scratch_types=[
            pltpu.VMEM_SHARED((N, D_HID), jnp.float32),
            pltpu.VMEM_SHARED((N, DEG_W), jnp.float32),
            pltpu.VMEM((NCHUNK, CHUNK), jnp.int32),
            pltpu.VMEM((NCHUNK, CHUNK), jnp.int32),
            pltpu.VMEM((CHUNK, D_HID), jnp.float32),
            pltpu.VMEM((CHUNK, D_HID), jnp.float32),
            pltpu.VMEM((CHUNK, DEG_W), jnp.float32),
            pltpu.SemaphoreType.DMA,
            pltpu.SemaphoreType.DMA,
            pltpu.SemaphoreType.DMA,
            pltpu.SemaphoreType.DMA,
        ],
    )(_sc_agg_body)
    return f(h, src3, dst3, z64, z16, ones)


def _fin_body(agg_ref, deg_ref, w_ref, b_ref, a_ref, o_ref):
    s = agg_ref[0] + agg_ref[1]
    d = deg_ref[0, :, 0:1] + deg_ref[1, :, 0:1]
    m = s / jnp.maximum(d, 1.0)
    p = jnp.where(m >= 0, m, a_ref[...] * m)
    o_ref[...] = (
        jnp.dot(p, w_ref[...], preferred_element_type=jnp.float32) + b_ref[...]
    )


def _stage3(aggp, degp, W2, b2, a):
    B = 1000
    a_row = jnp.full((1, D_HID), a, jnp.float32)
    return pl.pallas_call(
        _fin_body,
        grid=(N // B,),
        in_specs=[
            pl.BlockSpec((NC, B, D_HID), lambda i: (0, i, 0)),
            pl.BlockSpec((NC, B, DEG_W), lambda i: (0, i, 0)),
            pl.BlockSpec((D_HID, D_HID), lambda i: (0, 0)),
            pl.BlockSpec((1, D_HID), lambda i: (0, 0)),
            pl.BlockSpec((1, D_HID), lambda i: (0, 0)),
        ],
        out_specs=pl.BlockSpec((B, D_HID), lambda i: (i, 0)),
        out_shape=jax.ShapeDtypeStruct((N, D_HID), jnp.float32),
    )(aggp, degp, W2, b2.reshape(1, D_HID), a_row)


def kernel(x, edge_index, W1, b1, W2, b2, a):
    h = _stage1(x, W1, b1)
    aggp, degp = _stage2(h, edge_index[0], edge_index[1])
    return _stage3(aggp, degp, W2, b2, a)


# speedup vs baseline: 12.3371x; 1.0469x over previous
import functools

import jax
import jax.numpy as jnp
from jax import lax
from jax.experimental import pallas as pl
from jax.experimental.pallas import tpu as pltpu
from jax.experimental.pallas import tpu_sc as plsc

N = 10000
E = 320000
D_IN = 128
D_HID = 64

NC = 2
NS = 16
NW = NC * NS
E_PER_W = E // NW
CHUNK = 100
NCHUNK = E_PER_W // CHUNK
NCHUNK2 = NCHUNK // 2
ROWS_PER_TILE = 624
ROWS_TAIL = N - ROWS_PER_TILE * NS
DEG_W = 16


def _mm1_body(x_ref, w_ref, b_ref, o_ref):
    o_ref[...] = (
        jnp.dot(x_ref[...], w_ref[...], preferred_element_type=jnp.float32)
        + b_ref[...]
    )


def _stage1(x, W1, b1):
    B = 1000
    return pl.pallas_call(
        _mm1_body,
        grid=(N // B,),
        in_specs=[
            pl.BlockSpec((B, D_IN), lambda i: (i, 0)),
            pl.BlockSpec((D_IN, D_HID), lambda i: (0, 0)),
            pl.BlockSpec((1, D_HID), lambda i: (0, 0)),
        ],
        out_specs=pl.BlockSpec((B, D_HID), lambda i: (i, 0)),
        out_shape=jax.ShapeDtypeStruct((N, D_HID), jnp.float32),
    )(x, W1, b1.reshape(1, D_HID))


def _sc_agg_body(h_hbm, edges_hbm, z64_hbm, z16_hbm, ones_hbm,
                 agg_out, deg_out,
                 agg_sh, deg_sh, src_v, dst_v, rows0, rows1, ones_v,
                 sem_g0, sem_g1, sem_s0, sem_s1):
    cid = lax.axis_index("c")
    sid = lax.axis_index("s")
    wid = sid * NC + cid

    rbase = sid * ROWS_PER_TILE
    pltpu.sync_copy(z64_hbm.at[pl.ds(rbase, ROWS_PER_TILE)],
                    agg_sh.at[pl.ds(rbase, ROWS_PER_TILE)])
    pltpu.sync_copy(z16_hbm.at[pl.ds(rbase, ROWS_PER_TILE)],
                    deg_sh.at[pl.ds(rbase, ROWS_PER_TILE)])

    @pl.when(sid == NS - 1)
    def _zero_tail():
        tb = ROWS_PER_TILE * NS
        pltpu.sync_copy(z64_hbm.at[pl.ds(tb, ROWS_TAIL)],
                        agg_sh.at[pl.ds(tb, ROWS_TAIL)])
        pltpu.sync_copy(z16_hbm.at[pl.ds(tb, ROWS_TAIL)],
                        deg_sh.at[pl.ds(tb, ROWS_TAIL)])

    pltpu.sync_copy(edges_hbm.at[wid], src_v)
    pltpu.sync_copy(edges_hbm.at[NW + wid], dst_v)
    pltpu.sync_copy(ones_hbm, ones_v)
    plsc.subcore_barrier()

    def _fire_gather(j, buf, sem):
        pltpu.async_copy(h_hbm.at[src_v.at[j]], buf, sem)

    def _wait_gather(j, buf, sem):
        pltpu.make_async_copy(h_hbm.at[src_v.at[j]], buf, sem).wait()

    def _fire_scatter(j, buf, sem):
        pltpu.async_copy(buf, agg_sh.at[dst_v.at[j]], sem, add=True)
        pltpu.async_copy(ones_v, deg_sh.at[dst_v.at[j]], sem, add=True)

    def _wait_scatter(j, buf, sem):
        pltpu.make_async_copy(buf, agg_sh.at[dst_v.at[j]], sem).wait()
        pltpu.make_async_copy(ones_v, deg_sh.at[dst_v.at[j]], sem).wait()

    _fire_gather(0, rows0, sem_g0)

    def body(jj, carry):
        j0 = 2 * jj
        j1 = j0 + 1

        @pl.when(jj > 0)
        def _():
            _wait_scatter(j1, rows1, sem_s1)

        _fire_gather(j1, rows1, sem_g1)
        _wait_gather(j0, rows0, sem_g0)
        _fire_scatter(j0, rows0, sem_s0)
        _wait_scatter(j0, rows0, sem_s0)

        @pl.when(jj + 1 < NCHUNK2)
        def _():
            _fire_gather(j0 + 2, rows0, sem_g0)

        _wait_gather(j1, rows1, sem_g1)
        _fire_scatter(j1, rows1, sem_s1)
        return carry

    lax.fori_loop(0, NCHUNK2, body, 0)
    _wait_scatter(NCHUNK - 1, rows1, sem_s1)
    plsc.subcore_barrier()

    pltpu.sync_copy(agg_sh.at[pl.ds(rbase, ROWS_PER_TILE)],
                    agg_out.at[cid, pl.ds(rbase, ROWS_PER_TILE)])
    pltpu.sync_copy(deg_sh.at[pl.ds(rbase, ROWS_PER_TILE)],
                    deg_out.at[cid, pl.ds(rbase, ROWS_PER_TILE)])

    @pl.when(sid == NS - 1)
    def _publish_tail():
        tb = ROWS_PER_TILE * NS
        pltpu.sync_copy(agg_sh.at[pl.ds(tb, ROWS_TAIL)],
                        agg_out.at[cid, pl.ds(tb, ROWS_TAIL)])
        pltpu.sync_copy(deg_sh.at[pl.ds(tb, ROWS_TAIL)],
                        deg_out.at[cid, pl.ds(tb, ROWS_TAIL)])


def _stage2(h, edge_index):
    edges3 = edge_index.reshape(2 * NW, NCHUNK, CHUNK)
    z64 = jnp.zeros((N, D_HID), jnp.float32)
    z16 = jnp.zeros((N, DEG_W), jnp.float32)
    ones = jnp.ones((CHUNK, DEG_W), jnp.float32)
    mesh = plsc.VectorSubcoreMesh(core_axis_name="c", subcore_axis_name="s")
    f = functools.partial(
        pl.kernel,
        out_type=[
            jax.ShapeDtypeStruct((NC, N, D_HID), jnp.float32),
            jax.ShapeDtypeStruct((NC, N, DEG_W), jnp.float32),
        ],
        mesh=mesh,
        compiler_params=pltpu.CompilerParams(use_tc_tiling_on_sc=False),
        scratch_types=[
            pltpu.VMEM_SHARED((N, D_HID), jnp.float32),
            pltpu.VMEM_SHARED((N, DEG_W), jnp.float32),
            pltpu.VMEM((NCHUNK, CHUNK), jnp.int32),
            pltpu.VMEM((NCHUNK, CHUNK), jnp.int32),
            pltpu.VMEM((CHUNK, D_HID), jnp.float32),
            pltpu.VMEM((CHUNK, D_HID), jnp.float32),
            pltpu.VMEM((CHUNK, DEG_W), jnp.float32),
            pltpu.SemaphoreType.DMA,
            pltpu.SemaphoreType.DMA,
            pltpu.SemaphoreType.DMA,
            pltpu.SemaphoreType.DMA,
        ],
    )(_sc_agg_body)
    return f(h, edges3, z64, z16, ones)


def _fin_body(agg_ref, deg_ref, w_ref, b_ref, a_ref, o_ref):
    s = agg_ref[0] + agg_ref[1]
    d = deg_ref[0, :, 0:1] + deg_ref[1, :, 0:1]
    m = s / jnp.maximum(d, 1.0)
    p = jnp.where(m >= 0, m, a_ref[...] * m)
    o_ref[...] = (
        jnp.dot(p, w_ref[...], preferred_element_type=jnp.float32) + b_ref[...]
    )


def _stage3(aggp, degp, W2, b2, a):
    B = 1000
    a_row = jnp.full((1, D_HID), a, jnp.float32)
    return pl.pallas_call(
        _fin_body,
        grid=(N // B,),
        in_specs=[
            pl.BlockSpec((NC, B, D_HID), lambda i: (0, i, 0)),
            pl.BlockSpec((NC, B, DEG_W), lambda i: (0, i, 0)),
            pl.BlockSpec((D_HID, D_HID), lambda i: (0, 0)),
            pl.BlockSpec((1, D_HID), lambda i: (0, 0)),
            pl.BlockSpec((1, D_HID), lambda i: (0, 0)),
        ],
        out_specs=pl.BlockSpec((B, D_HID), lambda i: (i, 0)),
        out_shape=jax.ShapeDtypeStruct((N, D_HID), jnp.float32),
    )(aggp, degp, W2, b2.reshape(1, D_HID), a_row)


def kernel(x, edge_index, W1, b1, W2, b2, a):
    h = _stage1(x, W1, b1)
    aggp, degp = _stage2(h, edge_index)
    return _stage3(aggp, degp, W2, b2, a)


# speedup vs baseline: 12.9081x; 1.0463x over previous
import functools

import jax
import jax.numpy as jnp
from jax import lax
from jax.experimental import pallas as pl
from jax.experimental.pallas import tpu as pltpu
from jax.experimental.pallas import tpu_sc as plsc

N = 10000
E = 320000
D_IN = 128
D_HID = 64

NC = 2
NS = 16
NW = NC * NS
E_PER_W = E // NW
CHUNK = 100
NCHUNK = E_PER_W // CHUNK
NCHUNK2 = NCHUNK // 2
ROWS_PER_TILE = 624
ROWS_TAIL = N - ROWS_PER_TILE * NS
DEG_W = 16
RB = ROWS_PER_TILE // 16


def _mm1_body(x_ref, w_ref, b_ref, o_ref):
    o_ref[...] = (
        jnp.dot(x_ref[...], w_ref[...], preferred_element_type=jnp.float32)
        + b_ref[...]
    )


def _stage1(x, W1, b1):
    B = 1000
    x2 = x.reshape(N // 2, 2 * D_IN)
    wbd = jnp.zeros((2 * D_IN, 2 * D_HID), jnp.float32)
    wbd = wbd.at[:D_IN, :D_HID].set(W1).at[D_IN:, D_HID:].set(W1)
    bbd = jnp.concatenate([b1, b1]).reshape(1, 2 * D_HID)
    h2 = pl.pallas_call(
        _mm1_body,
        grid=(N // 2 // B,),
        in_specs=[
            pl.BlockSpec((B, 2 * D_IN), lambda i: (i, 0)),
            pl.BlockSpec((2 * D_IN, 2 * D_HID), lambda i: (0, 0)),
            pl.BlockSpec((1, 2 * D_HID), lambda i: (0, 0)),
        ],
        out_specs=pl.BlockSpec((B, 2 * D_HID), lambda i: (i, 0)),
        out_shape=jax.ShapeDtypeStruct((N // 2, 2 * D_HID), jnp.float32),
    )(x2, wbd, bbd)
    return h2.reshape(N, D_HID)


def _sc_agg_body(h_hbm, edges_hbm, z64_hbm, z16_hbm, ones_hbm,
                 agg_out, deg_out,
                 agg_sh, deg_sh, src_v, dst_v, rows0, rows1, ones_v,
                 sem_g0, sem_g1, sem_s0, sem_s1):
    cid = lax.axis_index("c")
    sid = lax.axis_index("s")
    wid = sid * NC + cid

    rbase = sid * ROWS_PER_TILE
    pltpu.sync_copy(z64_hbm.at[pl.ds(rbase, ROWS_PER_TILE)],
                    agg_sh.at[pl.ds(rbase, ROWS_PER_TILE)])
    pltpu.sync_copy(z16_hbm.at[pl.ds(rbase, ROWS_PER_TILE)],
                    deg_sh.at[pl.ds(rbase, ROWS_PER_TILE)])

    @pl.when(sid == NS - 1)
    def _zero_tail():
        tb = ROWS_PER_TILE * NS
        pltpu.sync_copy(z64_hbm.at[pl.ds(tb, ROWS_TAIL)],
                        agg_sh.at[pl.ds(tb, ROWS_TAIL)])
        pltpu.sync_copy(z16_hbm.at[pl.ds(tb, ROWS_TAIL)],
                        deg_sh.at[pl.ds(tb, ROWS_TAIL)])

    pltpu.sync_copy(edges_hbm.at[wid], src_v)
    pltpu.sync_copy(edges_hbm.at[NW + wid], dst_v)
    pltpu.sync_copy(ones_hbm, ones_v)
    plsc.subcore_barrier()

    def _fire_gather(j, buf, sem):
        pltpu.async_copy(h_hbm.at[src_v.at[j]], buf, sem)

    def _wait_gather(j, buf, sem):
        pltpu.make_async_copy(h_hbm.at[src_v.at[j]], buf, sem).wait()

    def _fire_scatter(j, buf, sem):
        pltpu.async_copy(buf, agg_sh.at[dst_v.at[j]], sem, add=True)
        pltpu.async_copy(ones_v, deg_sh.at[dst_v.at[j]], sem, add=True)

    def _wait_scatter(j, buf, sem):
        pltpu.make_async_copy(buf, agg_sh.at[dst_v.at[j]], sem).wait()
        pltpu.make_async_copy(ones_v, deg_sh.at[dst_v.at[j]], sem).wait()

    _fire_gather(0, rows0, sem_g0)

    def body(jj, carry):
        j0 = 2 * jj
        j1 = j0 + 1

        @pl.when(jj > 0)
        def _():
            _wait_scatter(j1, rows1, sem_s1)

        _fire_gather(j1, rows1, sem_g1)
        _wait_gather(j0, rows0, sem_g0)
        _fire_scatter(j0, rows0, sem_s0)
        _wait_scatter(j0, rows0, sem_s0)

        @pl.when(jj + 1 < NCHUNK2)
        def _():
            _fire_gather(j0 + 2, rows0, sem_g0)

        _wait_gather(j1, rows1, sem_g1)
        _fire_scatter(j1, rows1, sem_s1)
        return carry

    lax.fori_loop(0, NCHUNK2, body, 0)
    _wait_scatter(NCHUNK - 1, rows1, sem_s1)
    plsc.subcore_barrier()

    pltpu.sync_copy(agg_sh.at[pl.ds(rbase, ROWS_PER_TILE)],
                    agg_out.at[pl.ds(rbase, ROWS_PER_TILE),
                               pl.ds(cid * D_HID, D_HID)])

    pltpu.sync_copy(deg_sh.at[pl.ds(rbase, ROWS_PER_TILE)],
                    deg_out.at[cid, pl.ds(rbase, ROWS_PER_TILE)])

    @pl.when(sid == NS - 1)
    def _publish_tail():
        tb = ROWS_PER_TILE * NS
        pltpu.sync_copy(agg_sh.at[pl.ds(tb, ROWS_TAIL)],
                        agg_out.at[pl.ds(tb, ROWS_TAIL),
                                   pl.ds(cid * D_HID, D_HID)])
        pltpu.sync_copy(deg_sh.at[pl.ds(tb, ROWS_TAIL)],
                        deg_out.at[cid, pl.ds(tb, ROWS_TAIL)])


def _stage2(h, edge_index):
    edges3 = edge_index.reshape(2 * NW, NCHUNK, CHUNK)
    z64 = jnp.zeros((N, D_HID), jnp.float32)
    z16 = jnp.zeros((N, DEG_W), jnp.float32)
    ones = jnp.ones((CHUNK, DEG_W), jnp.float32)
    mesh = plsc.VectorSubcoreMesh(core_axis_name="c", subcore_axis_name="s")
    f = functools.partial(
        pl.kernel,
        out_type=[
            jax.ShapeDtypeStruct((N, 2 * D_HID), jnp.float32),
            jax.ShapeDtypeStruct((NC, N, DEG_W), jnp.float32),
        ],
        mesh=mesh,
        compiler_params=pltpu.CompilerParams(use_tc_tiling_on_sc=False),
        scratch_types=[
            pltpu.VMEM_SHARED((N, D_HID), jnp.float32),
            pltpu.VMEM_SHARED((N, DEG_W), jnp.float32),
            pltpu.VMEM((NCHUNK, CHUNK), jnp.int32),
            pltpu.VMEM((NCHUNK, CHUNK), jnp.int32),
            pltpu.VMEM((CHUNK, D_HID), jnp.float32),
            pltpu.VMEM((CHUNK, D_HID), jnp.float32),
            pltpu.VMEM((CHUNK, DEG_W), jnp.float32),
            pltpu.SemaphoreType.DMA,
            pltpu.SemaphoreType.DMA,
            pltpu.SemaphoreType.DMA,
            pltpu.SemaphoreType.DMA,
        ],
    )(_sc_agg_body)
    return f(h, edges3, z64, z16, ones)


def _fin_body(agg_ref, deg_ref, w_ref, b_ref, a_ref, o_ref):
    s = agg_ref[:, :D_HID] + agg_ref[:, D_HID:]
    d = deg_ref[0, :, 0:1] + deg_ref[1, :, 0:1]
    m = s / jnp.maximum(d, 1.0)
    p = jnp.where(m >= 0, m, a_ref[...] * m)
    o_ref[...] = (
        jnp.dot(p, w_ref[...], preferred_element_type=jnp.float32) + b_ref[...]
    )


def _stage3(aggc, degp, W2, b2, a):
    B = 1000
    a_row = jnp.full((1, D_HID), a, jnp.float32)
    return pl.pallas_call(
        _fin_body,
        grid=(N // B,),
        in_specs=[
            pl.BlockSpec((B, 2 * D_HID), lambda i: (i, 0)),
            pl.BlockSpec((NC, B, DEG_W), lambda i: (0, i, 0)),
            pl.BlockSpec((D_HID, D_HID), lambda i: (0, 0)),
            pl.BlockSpec((1, D_HID), lambda i: (0, 0)),
            pl.BlockSpec((1, D_HID), lambda i: (0, 0)),
        ],
        out_specs=pl.BlockSpec((B, D_HID), lambda i: (i, 0)),
        out_shape=jax.ShapeDtypeStruct((N, D_HID), jnp.float32),
    )(aggc, degp, W2, b2.reshape(1, D_HID), a_row)


def kernel(x, edge_index, W1, b1, W2, b2, a):
    h = _stage1(x, W1, b1)
    aggc, degp = _stage2(h, edge_index)
    return _stage3(aggc, degp, W2, b2, a)


# speedup vs baseline: 14.3486x; 1.1116x over previous
import functools

import jax
import jax.numpy as jnp
from jax import lax
from jax.experimental import pallas as pl
from jax.experimental.pallas import tpu as pltpu
from jax.experimental.pallas import tpu_sc as plsc

N = 10000
E = 320000
D_IN = 128
D_HID = 64

NC = 2
NS = 16
NW = NC * NS
CHUNK = 128
EROWS = E // CHUNK
ROWS_BASE = EROWS // NW
ROWS_EXTRA = EROWS - ROWS_BASE * NW
NCHUNK2 = ROWS_BASE // 2
ROWS_PER_TILE = 624
ROWS_TAIL = N - ROWS_PER_TILE * NS
DEG_W = 16
RB = ROWS_PER_TILE // 16


def _mm1_body(x_ref, w_ref, b_ref, o_ref):
    o_ref[...] = (
        jnp.dot(x_ref[...], w_ref[...], preferred_element_type=jnp.float32)
        + b_ref[...]
    )


def _stage1(x, W1, b1):
    B = 1000
    x2 = x.reshape(N // 2, 2 * D_IN)
    wbd = jnp.zeros((2 * D_IN, 2 * D_HID), jnp.float32)
    wbd = wbd.at[:D_IN, :D_HID].set(W1).at[D_IN:, D_HID:].set(W1)
    bbd = jnp.concatenate([b1, b1]).reshape(1, 2 * D_HID)
    h2 = pl.pallas_call(
        _mm1_body,
        grid=(N // 2 // B,),
        in_specs=[
            pl.BlockSpec((B, 2 * D_IN), lambda i: (i, 0)),
            pl.BlockSpec((2 * D_IN, 2 * D_HID), lambda i: (0, 0)),
            pl.BlockSpec((1, 2 * D_HID), lambda i: (0, 0)),
        ],
        out_specs=pl.BlockSpec((B, 2 * D_HID), lambda i: (i, 0)),
        out_shape=jax.ShapeDtypeStruct((N // 2, 2 * D_HID), jnp.float32),
    )(x2, wbd, bbd)
    return h2.reshape(N, D_HID)


def _sc_agg_body(h_hbm, edges_hbm, z64_hbm, z16_hbm, ones_hbm,
                 agg_out, deg_out,
                 agg_sh, deg_sh, src_v, dst_v, rows0, rows1, ones_v,
                 sem_g0, sem_g1, sem_s0, sem_s1):
    cid = lax.axis_index("c")
    sid = lax.axis_index("s")
    wid = sid * NC + cid

    rbase = sid * ROWS_PER_TILE
    pltpu.sync_copy(z64_hbm.at[pl.ds(rbase, ROWS_PER_TILE)],
                    agg_sh.at[pl.ds(rbase, ROWS_PER_TILE)])
    pltpu.sync_copy(z16_hbm.at[pl.ds(rbase, ROWS_PER_TILE)],
                    deg_sh.at[pl.ds(rbase, ROWS_PER_TILE)])

    @pl.when(sid == NS - 1)
    def _zero_tail():
        tb = ROWS_PER_TILE * NS
        pltpu.sync_copy(z64_hbm.at[pl.ds(tb, ROWS_TAIL)],
                        agg_sh.at[pl.ds(tb, ROWS_TAIL)])
        pltpu.sync_copy(z16_hbm.at[pl.ds(tb, ROWS_TAIL)],
                        deg_sh.at[pl.ds(tb, ROWS_TAIL)])

    row_off = ROWS_BASE * wid + jnp.minimum(wid, ROWS_EXTRA)
    pltpu.sync_copy(edges_hbm.at[0, pl.ds(row_off, ROWS_BASE)],
                    src_v.at[pl.ds(0, ROWS_BASE)])
    pltpu.sync_copy(edges_hbm.at[1, pl.ds(row_off, ROWS_BASE)],
                    dst_v.at[pl.ds(0, ROWS_BASE)])

    @pl.when(wid < ROWS_EXTRA)
    def _stage_extra():
        pltpu.sync_copy(edges_hbm.at[0, pl.ds(row_off + ROWS_BASE, 1)],
                        src_v.at[pl.ds(ROWS_BASE, 1)])
        pltpu.sync_copy(edges_hbm.at[1, pl.ds(row_off + ROWS_BASE, 1)],
                        dst_v.at[pl.ds(ROWS_BASE, 1)])

    pltpu.sync_copy(ones_hbm, ones_v)
    plsc.subcore_barrier()

    def _fire_gather(j, buf, sem):
        pltpu.async_copy(h_hbm.at[src_v.at[j]], buf, sem)

    def _wait_gather(j, buf, sem):
        pltpu.make_async_copy(h_hbm.at[src_v.at[j]], buf, sem).wait()

    def _fire_scatter(j, buf, sem):
        pltpu.async_copy(buf, agg_sh.at[dst_v.at[j]], sem, add=True)
        pltpu.async_copy(ones_v, deg_sh.at[dst_v.at[j]], sem, add=True)

    def _wait_scatter(j, buf, sem):
        pltpu.make_async_copy(buf, agg_sh.at[dst_v.at[j]], sem).wait()
        pltpu.make_async_copy(ones_v, deg_sh.at[dst_v.at[j]], sem).wait()

    _fire_gather(0, rows0, sem_g0)

    def body(jj, carry):
        j0 = 2 * jj
        j1 = j0 + 1

        @pl.when(jj > 0)
        def _():
            _wait_scatter(j1, rows1, sem_s1)

        _fire_gather(j1, rows1, sem_g1)
        _wait_gather(j0, rows0, sem_g0)
        _fire_scatter(j0, rows0, sem_s0)
        _wait_scatter(j0, rows0, sem_s0)

        @pl.when(jj + 1 < NCHUNK2)
        def _():
            _fire_gather(j0 + 2, rows0, sem_g0)

        _wait_gather(j1, rows1, sem_g1)
        _fire_scatter(j1, rows1, sem_s1)
        return carry

    lax.fori_loop(0, NCHUNK2, body, 0)
    _wait_scatter(ROWS_BASE - 1, rows1, sem_s1)

    @pl.when(wid < ROWS_EXTRA)
    def _extra_chunk():
        _fire_gather(ROWS_BASE, rows0, sem_g0)
        _wait_gather(ROWS_BASE, rows0, sem_g0)
        _fire_scatter(ROWS_BASE, rows0, sem_s0)
        _wait_scatter(ROWS_BASE, rows0, sem_s0)

    plsc.subcore_barrier()

    pltpu.sync_copy(agg_sh.at[pl.ds(rbase, ROWS_PER_TILE)],
                    agg_out.at[pl.ds(rbase, ROWS_PER_TILE),
                               pl.ds(cid * D_HID, D_HID)])

    pltpu.sync_copy(deg_sh.at[pl.ds(rbase, ROWS_PER_TILE)],
                    deg_out.at[cid, pl.ds(rbase, ROWS_PER_TILE)])

    @pl.when(sid == NS - 1)
    def _publish_tail():
        tb = ROWS_PER_TILE * NS
        pltpu.sync_copy(agg_sh.at[pl.ds(tb, ROWS_TAIL)],
                        agg_out.at[pl.ds(tb, ROWS_TAIL),
                                   pl.ds(cid * D_HID, D_HID)])
        pltpu.sync_copy(deg_sh.at[pl.ds(tb, ROWS_TAIL)],
                        deg_out.at[cid, pl.ds(tb, ROWS_TAIL)])


def _stage2(h, edge_index):
    edges3 = edge_index.reshape(2, EROWS, CHUNK)
    z64 = jnp.zeros((N, D_HID), jnp.float32)
    z16 = jnp.zeros((N, DEG_W), jnp.float32)
    ones = jnp.ones((CHUNK, DEG_W), jnp.float32)
    mesh = plsc.VectorSubcoreMesh(core_axis_name="c", subcore_axis_name="s")
    f = functools.partial(
        pl.kernel,
        out_type=[
            jax.ShapeDtypeStruct((N, 2 * D_HID), jnp.float32),
            jax.ShapeDtypeStruct((NC, N, DEG_W), jnp.float32),
        ],
        mesh=mesh,
        compiler_params=pltpu.CompilerParams(use_tc_tiling_on_sc=False),
        scratch_types=[
            pltpu.VMEM_SHARED((N, D_HID), jnp.float32),
            pltpu.VMEM_SHARED((N, DEG_W), jnp.float32),
            pltpu.VMEM((ROWS_BASE + 1, CHUNK), jnp.int32),
            pltpu.VMEM((ROWS_BASE + 1, CHUNK), jnp.int32),
            pltpu.VMEM((CHUNK, D_HID), jnp.float32),
            pltpu.VMEM((CHUNK, D_HID), jnp.float32),
            pltpu.VMEM((CHUNK, DEG_W), jnp.float32),
            pltpu.SemaphoreType.DMA,
            pltpu.SemaphoreType.DMA,
            pltpu.SemaphoreType.DMA,
            pltpu.SemaphoreType.DMA,
        ],
    )(_sc_agg_body)
    return f(h, edges3, z64, z16, ones)


def _fin_body(agg_ref, deg_ref, w_ref, b_ref, a_ref, o_ref):
    s = agg_ref[:, :D_HID] + agg_ref[:, D_HID:]
    d = deg_ref[0, :, 0:1] + deg_ref[1, :, 0:1]
    m = s / jnp.maximum(d, 1.0)
    p = jnp.where(m >= 0, m, a_ref[...] * m)
    o_ref[...] = (
        jnp.dot(p, w_ref[...], preferred_element_type=jnp.float32) + b_ref[...]
    )


def _stage3(aggc, degp, W2, b2, a):
    B = 1000
    a_row = jnp.full((1, D_HID), a, jnp.float32)
    return pl.pallas_call(
        _fin_body,
        grid=(N // B,),
        in_specs=[
            pl.BlockSpec((B, 2 * D_HID), lambda i: (i, 0)),
            pl.BlockSpec((NC, B, DEG_W), lambda i: (0, i, 0)),
            pl.BlockSpec((D_HID, D_HID), lambda i: (0, 0)),
            pl.BlockSpec((1, D_HID), lambda i: (0, 0)),
            pl.BlockSpec((1, D_HID), lambda i: (0, 0)),
        ],
        out_specs=pl.BlockSpec((B, D_HID), lambda i: (i, 0)),
        out_shape=jax.ShapeDtypeStruct((N, D_HID), jnp.float32),
    )(aggc, degp, W2, b2.reshape(1, D_HID), a_row)


def kernel(x, edge_index, W1, b1, W2, b2, a):
    h = _stage1(x, W1, b1)
    aggc, degp = _stage2(h, edge_index)
    return _stage3(aggc, degp, W2, b2, a)


# speedup vs baseline: 14.7917x; 1.0309x over previous
import functools

import jax
import jax.numpy as jnp
from jax import lax
from jax.experimental import pallas as pl
from jax.experimental.pallas import tpu as pltpu
from jax.experimental.pallas import tpu_sc as plsc

N = 10000
E = 320000
D_IN = 128
D_HID = 64

NC = 2
NS = 16
NW = NC * NS
CHUNK = 128
EROWS = E // CHUNK
ROWS_BASE = EROWS // NW
ROWS_EXTRA = EROWS - ROWS_BASE * NW
NCHUNK2 = ROWS_BASE // 2
ROWS_PER_TILE = 624
ROWS_TAIL = N - ROWS_PER_TILE * NS
DEG_W = 16
RB = ROWS_PER_TILE // 16


def _mm1_body(x_ref, w_ref, b_ref, o_ref):
    o_ref[...] = (
        jnp.dot(x_ref[...], w_ref[...], preferred_element_type=jnp.float32)
        + b_ref[...]
    )


def _stage1(x, W1, b1):
    B = 1000
    x2 = x.reshape(N // 2, 2 * D_IN)
    wbd = jnp.zeros((2 * D_IN, 2 * D_HID), jnp.float32)
    wbd = wbd.at[:D_IN, :D_HID].set(W1).at[D_IN:, D_HID:].set(W1)
    bbd = jnp.concatenate([b1, b1]).reshape(1, 2 * D_HID)
    h2 = pl.pallas_call(
        _mm1_body,
        grid=(N // 2 // B,),
        in_specs=[
            pl.BlockSpec((B, 2 * D_IN), lambda i: (i, 0)),
            pl.BlockSpec((2 * D_IN, 2 * D_HID), lambda i: (0, 0)),
            pl.BlockSpec((1, 2 * D_HID), lambda i: (0, 0)),
        ],
        out_specs=pl.BlockSpec((B, 2 * D_HID), lambda i: (i, 0)),
        out_shape=jax.ShapeDtypeStruct((N // 2, 2 * D_HID), jnp.float32),
    )(x2, wbd, bbd)
    return h2.reshape(N, D_HID)


def _sc_agg_body(h_hbm, edges_hbm, z64_hbm, z16_hbm, ones_hbm,
                 agg_out, deg_out,
                 agg_sh, deg_sh, src_v, dst_v, rows0, rows1, ones_v,
                 sem_g0, sem_g1, sem_s0, sem_s1):
    cid = lax.axis_index("c")
    sid = lax.axis_index("s")
    wid = sid * NC + cid

    rbase = sid * ROWS_PER_TILE
    pltpu.sync_copy(z64_hbm.at[pl.ds(rbase, ROWS_PER_TILE)],
                    agg_sh.at[pl.ds(rbase, ROWS_PER_TILE)])
    pltpu.sync_copy(z16_hbm.at[pl.ds(rbase, ROWS_PER_TILE)],
                    deg_sh.at[pl.ds(rbase, ROWS_PER_TILE)])

    @pl.when(sid == NS - 1)
    def _zero_tail():
        tb = ROWS_PER_TILE * NS
        pltpu.sync_copy(z64_hbm.at[pl.ds(tb, ROWS_TAIL)],
                        agg_sh.at[pl.ds(tb, ROWS_TAIL)])
        pltpu.sync_copy(z16_hbm.at[pl.ds(tb, ROWS_TAIL)],
                        deg_sh.at[pl.ds(tb, ROWS_TAIL)])

    row_off = ROWS_BASE * wid + jnp.minimum(wid, ROWS_EXTRA)
    pltpu.sync_copy(edges_hbm.at[0, pl.ds(row_off, ROWS_BASE)],
                    src_v.at[pl.ds(0, ROWS_BASE)])
    pltpu.sync_copy(edges_hbm.at[1, pl.ds(row_off, ROWS_BASE)],
                    dst_v.at[pl.ds(0, ROWS_BASE)])

    @pl.when(wid < ROWS_EXTRA)
    def _stage_extra():
        pltpu.sync_copy(edges_hbm.at[0, pl.ds(row_off + ROWS_BASE, 1)],
                        src_v.at[pl.ds(ROWS_BASE, 1)])
        pltpu.sync_copy(edges_hbm.at[1, pl.ds(row_off + ROWS_BASE, 1)],
                        dst_v.at[pl.ds(ROWS_BASE, 1)])

    pltpu.sync_copy(ones_hbm, ones_v)
    plsc.subcore_barrier()

    def _fire_gather(j, buf, sem):
        pltpu.async_copy(h_hbm.at[src_v.at[j]], buf, sem)

    def _wait_gather(j, buf, sem):
        pltpu.make_async_copy(h_hbm.at[src_v.at[j]], buf, sem).wait()

    def _fire_scatter(j, buf, sem):
        pltpu.async_copy(buf, agg_sh.at[dst_v.at[j]], sem, add=True)
        pltpu.async_copy(ones_v, deg_sh.at[dst_v.at[j]], sem, add=True)

    def _wait_scatter(j, buf, sem):
        pltpu.make_async_copy(buf, agg_sh.at[dst_v.at[j]], sem).wait()
        pltpu.make_async_copy(ones_v, deg_sh.at[dst_v.at[j]], sem).wait()

    _fire_gather(0, rows0, sem_g0)

    def body(jj, carry):
        j0 = 2 * jj
        j1 = j0 + 1

        @pl.when(jj > 0)
        def _():
            _wait_scatter(j1, rows1, sem_s1)

        _fire_gather(j1, rows1, sem_g1)
        _wait_gather(j0, rows0, sem_g0)
        _fire_scatter(j0, rows0, sem_s0)
        _wait_scatter(j0, rows0, sem_s0)

        @pl.when(jj + 1 < NCHUNK2)
        def _():
            _fire_gather(j0 + 2, rows0, sem_g0)

        _wait_gather(j1, rows1, sem_g1)
        _fire_scatter(j1, rows1, sem_s1)
        return carry

    lax.fori_loop(0, NCHUNK2, body, 0)
    _wait_scatter(ROWS_BASE - 1, rows1, sem_s1)

    @pl.when(wid < ROWS_EXTRA)
    def _extra_chunk():
        _fire_gather(ROWS_BASE, rows0, sem_g0)
        _wait_gather(ROWS_BASE, rows0, sem_g0)
        _fire_scatter(ROWS_BASE, rows0, sem_s0)
        _wait_scatter(ROWS_BASE, rows0, sem_s0)

    plsc.subcore_barrier()

    pltpu.sync_copy(agg_sh.at[pl.ds(rbase, ROWS_PER_TILE)],
                    agg_out.at[pl.ds(rbase, ROWS_PER_TILE),
                               pl.ds(cid * D_HID, D_HID)])

    pltpu.sync_copy(deg_sh.at[pl.ds(rbase, ROWS_PER_TILE)],
                    deg_out.at[pl.ds(rbase, ROWS_PER_TILE),
                               pl.ds(cid * DEG_W, DEG_W)])

    @pl.when(sid == NS - 1)
    def _publish_tail():
        tb = ROWS_PER_TILE * NS
        pltpu.sync_copy(agg_sh.at[pl.ds(tb, ROWS_TAIL)],
                        agg_out.at[pl.ds(tb, ROWS_TAIL),
                                   pl.ds(cid * D_HID, D_HID)])
        pltpu.sync_copy(deg_sh.at[pl.ds(tb, ROWS_TAIL)],
                        deg_out.at[pl.ds(tb, ROWS_TAIL),
                                   pl.ds(cid * DEG_W, DEG_W)])


def _stage2(h, edge_index):
    edges3 = edge_index.reshape(2, EROWS, CHUNK)
    z64 = jnp.zeros((N, D_HID), jnp.float32)
    z16 = jnp.zeros((N, DEG_W), jnp.float32)
    ones = jnp.ones((CHUNK, DEG_W), jnp.float32)
    mesh = plsc.VectorSubcoreMesh(core_axis_name="c", subcore_axis_name="s")
    f = functools.partial(
        pl.kernel,
        out_type=[
            jax.ShapeDtypeStruct((N, 2 * D_HID), jnp.float32),
            jax.ShapeDtypeStruct((N, 8 * DEG_W), jnp.float32),
        ],
        mesh=mesh,
        compiler_params=pltpu.CompilerParams(use_tc_tiling_on_sc=False),
        scratch_types=[
            pltpu.VMEM_SHARED((N, D_HID), jnp.float32),
            pltpu.VMEM_SHARED((N, DEG_W), jnp.float32),
            pltpu.VMEM((ROWS_BASE + 1, CHUNK), jnp.int32),
            pltpu.VMEM((ROWS_BASE + 1, CHUNK), jnp.int32),
            pltpu.VMEM((CHUNK, D_HID), jnp.float32),
            pltpu.VMEM((CHUNK, D_HID), jnp.float32),
            pltpu.VMEM((CHUNK, DEG_W), jnp.float32),
            pltpu.SemaphoreType.DMA,
            pltpu.SemaphoreType.DMA,
            pltpu.SemaphoreType.DMA,
            pltpu.SemaphoreType.DMA,
        ],
    )(_sc_agg_body)
    return f(h, edges3, z64, z16, ones)


def _fin_body(agg_ref, deg_ref, w_ref, b_ref, a_ref, o_ref):
    s = agg_ref[:, :D_HID] + agg_ref[:, D_HID:]
    d = deg_ref[:, 0:1] + deg_ref[:, DEG_W:DEG_W + 1]
    m = s / jnp.maximum(d, 1.0)
    p = jnp.where(m >= 0, m, a_ref[...] * m)
    o_ref[...] = (
        jnp.dot(p, w_ref[...], preferred_element_type=jnp.float32) + b_ref[...]
    )


def _stage3(aggc, degp, W2, b2, a):
    B = 1000
    a_row = jnp.full((1, D_HID), a, jnp.float32)
    return pl.pallas_call(
        _fin_body,
        grid=(N // B,),
        in_specs=[
            pl.BlockSpec((B, 2 * D_HID), lambda i: (i, 0)),
            pl.BlockSpec((B, 8 * DEG_W), lambda i: (i, 0)),
            pl.BlockSpec((D_HID, D_HID), lambda i: (0, 0)),
            pl.BlockSpec((1, D_HID), lambda i: (0, 0)),
            pl.BlockSpec((1, D_HID), lambda i: (0, 0)),
        ],
        out_specs=pl.BlockSpec((B, D_HID), lambda i: (i, 0)),
        out_shape=jax.ShapeDtypeStruct((N, D_HID), jnp.float32),
    )(aggc, degp, W2, b2.reshape(1, D_HID), a_row)


def kernel(x, edge_index, W1, b1, W2, b2, a):
    h = _stage1(x, W1, b1)
    aggc, degp = _stage2(h, edge_index)
    return _stage3(aggc, degp, W2, b2, a)


# speedup vs baseline: 15.3783x; 1.0397x over previous
import functools

import jax
import jax.numpy as jnp
from jax import lax
from jax.experimental import pallas as pl
from jax.experimental.pallas import tpu as pltpu
from jax.experimental.pallas import tpu_sc as plsc

N = 10000
E = 320000
D_IN = 128
D_HID = 64

NC = 2
NS = 16
NW = NC * NS
CHUNK = 128
EROWS = E // CHUNK
ROWS_BASE = EROWS // NW
ROWS_EXTRA = EROWS - ROWS_BASE * NW
NCHUNK3 = ROWS_BASE // 3
ROWS_PER_TILE = 624
ROWS_TAIL = N - ROWS_PER_TILE * NS
DEG_W = 16
RB = ROWS_PER_TILE // 16


def _mm1_body(x_ref, w_ref, b_ref, o_ref):
    o_ref[...] = (
        jnp.dot(x_ref[...], w_ref[...], preferred_element_type=jnp.float32)
        + b_ref[...]
    )


def _stage1(x, W1, b1):
    B = 1000
    x2 = x.reshape(N // 2, 2 * D_IN)
    wbd = jnp.zeros((2 * D_IN, 2 * D_HID), jnp.float32)
    wbd = wbd.at[:D_IN, :D_HID].set(W1).at[D_IN:, D_HID:].set(W1)
    bbd = jnp.concatenate([b1, b1]).reshape(1, 2 * D_HID)
    h2 = pl.pallas_call(
        _mm1_body,
        grid=(N // 2 // B,),
        in_specs=[
            pl.BlockSpec((B, 2 * D_IN), lambda i: (i, 0)),
            pl.BlockSpec((2 * D_IN, 2 * D_HID), lambda i: (0, 0)),
            pl.BlockSpec((1, 2 * D_HID), lambda i: (0, 0)),
        ],
        out_specs=pl.BlockSpec((B, 2 * D_HID), lambda i: (i, 0)),
        out_shape=jax.ShapeDtypeStruct((N // 2, 2 * D_HID), jnp.float32),
    )(x2, wbd, bbd)
    return h2.reshape(N, D_HID)


def _sc_agg_body(h_hbm, edges_hbm, z64_hbm, z16_hbm, ones_hbm,
                 agg_out, deg_out,
                 agg_sh, deg_sh, src_v, dst_v, rows0, rows1, rows2, ones_v,
                 sem_g0, sem_g1, sem_g2, sem_s0, sem_s1, sem_s2):
    cid = lax.axis_index("c")
    sid = lax.axis_index("s")
    wid = sid * NC + cid

    rbase = sid * ROWS_PER_TILE
    pltpu.sync_copy(z64_hbm.at[pl.ds(rbase, ROWS_PER_TILE)],
                    agg_sh.at[pl.ds(rbase, ROWS_PER_TILE)])
    pltpu.sync_copy(z16_hbm.at[pl.ds(rbase, ROWS_PER_TILE)],
                    deg_sh.at[pl.ds(rbase, ROWS_PER_TILE)])

    @pl.when(sid == NS - 1)
    def _zero_tail():
        tb = ROWS_PER_TILE * NS
        pltpu.sync_copy(z64_hbm.at[pl.ds(tb, ROWS_TAIL)],
                        agg_sh.at[pl.ds(tb, ROWS_TAIL)])
        pltpu.sync_copy(z16_hbm.at[pl.ds(tb, ROWS_TAIL)],
                        deg_sh.at[pl.ds(tb, ROWS_TAIL)])

    row_off = ROWS_BASE * wid + jnp.minimum(wid, ROWS_EXTRA)
    pltpu.sync_copy(edges_hbm.at[0, pl.ds(row_off, ROWS_BASE)],
                    src_v.at[pl.ds(0, ROWS_BASE)])
    pltpu.sync_copy(edges_hbm.at[1, pl.ds(row_off, ROWS_BASE)],
                    dst_v.at[pl.ds(0, ROWS_BASE)])

    @pl.when(wid < ROWS_EXTRA)
    def _stage_extra():
        pltpu.sync_copy(edges_hbm.at[0, pl.ds(row_off + ROWS_BASE, 1)],
                        src_v.at[pl.ds(ROWS_BASE, 1)])
        pltpu.sync_copy(edges_hbm.at[1, pl.ds(row_off + ROWS_BASE, 1)],
                        dst_v.at[pl.ds(ROWS_BASE, 1)])

    pltpu.sync_copy(ones_hbm, ones_v)
    plsc.subcore_barrier()

    def _fire_gather(j, buf, sem):
        pltpu.async_copy(h_hbm.at[src_v.at[j]], buf, sem)

    def _wait_gather(j, buf, sem):
        pltpu.make_async_copy(h_hbm.at[src_v.at[j]], buf, sem).wait()

    def _fire_scatter(j, buf, sem):
        pltpu.async_copy(buf, agg_sh.at[dst_v.at[j]], sem, add=True)
        pltpu.async_copy(ones_v, deg_sh.at[dst_v.at[j]], sem, add=True)

    def _wait_scatter(j, buf, sem):
        pltpu.make_async_copy(buf, agg_sh.at[dst_v.at[j]], sem).wait()
        pltpu.make_async_copy(ones_v, deg_sh.at[dst_v.at[j]], sem).wait()

    _fire_gather(0, rows0, sem_g0)

    def body(jj, carry):
        j0 = 3 * jj

        @pl.when(jj > 0)
        def _():
            _wait_scatter(j0 - 2, rows1, sem_s1)

        _fire_gather(j0 + 1, rows1, sem_g1)
        _wait_gather(j0, rows0, sem_g0)
        _fire_scatter(j0, rows0, sem_s0)

        @pl.when(jj > 0)
        def _():
            _wait_scatter(j0 - 1, rows2, sem_s2)

        _fire_gather(j0 + 2, rows2, sem_g2)
        _wait_gather(j0 + 1, rows1, sem_g1)
        _fire_scatter(j0 + 1, rows1, sem_s1)
        _wait_scatter(j0, rows0, sem_s0)

        @pl.when(jj + 1 < NCHUNK3)
        def _():
            _fire_gather(j0 + 3, rows0, sem_g0)

        _wait_gather(j0 + 2, rows2, sem_g2)
        _fire_scatter(j0 + 2, rows2, sem_s2)
        return carry

    lax.fori_loop(0, NCHUNK3, body, 0)
    _wait_scatter(ROWS_BASE - 2, rows1, sem_s1)
    _wait_scatter(ROWS_BASE - 1, rows2, sem_s2)

    @pl.when(wid < ROWS_EXTRA)
    def _extra_chunk():
        _fire_gather(ROWS_BASE, rows0, sem_g0)
        _wait_gather(ROWS_BASE, rows0, sem_g0)
        _fire_scatter(ROWS_BASE, rows0, sem_s0)
        _wait_scatter(ROWS_BASE, rows0, sem_s0)

    plsc.subcore_barrier()

    pltpu.sync_copy(agg_sh.at[pl.ds(rbase, ROWS_PER_TILE)],
                    agg_out.at[pl.ds(rbase, ROWS_PER_TILE),
                               pl.ds(cid * D_HID, D_HID)])

    pltpu.sync_copy(deg_sh.at[pl.ds(rbase, ROWS_PER_TILE)],
                    deg_out.at[pl.ds(rbase, ROWS_PER_TILE),
                               pl.ds(cid * DEG_W, DEG_W)])

    @pl.when(sid == NS - 1)
    def _publish_tail():
        tb = ROWS_PER_TILE * NS
        pltpu.sync_copy(agg_sh.at[pl.ds(tb, ROWS_TAIL)],
                        agg_out.at[pl.ds(tb, ROWS_TAIL),
                                   pl.ds(cid * D_HID, D_HID)])
        pltpu.sync_copy(deg_sh.at[pl.ds(tb, ROWS_TAIL)],
                        deg_out.at[pl.ds(tb, ROWS_TAIL),
                                   pl.ds(cid * DEG_W, DEG_W)])


def _stage2(h, edge_index):
    edges3 = edge_index.reshape(2, EROWS, CHUNK)
    z64 = jnp.zeros((N, D_HID), jnp.float32)
    z16 = jnp.zeros((N, DEG_W), jnp.float32)
    ones = jnp.ones((CHUNK, DEG_W), jnp.float32)
    mesh = plsc.VectorSubcoreMesh(core_axis_name="c", subcore_axis_name="s")
    f = functools.partial(
        pl.kernel,
        out_type=[
            jax.ShapeDtypeStruct((N, 2 * D_HID), jnp.float32),
            jax.ShapeDtypeStruct((N, 8 * DEG_W), jnp.float32),
        ],
        mesh=mesh,
        compiler_params=pltpu.CompilerParams(use_tc_tiling_on_sc=False),
        scratch_types=[
            pltpu.VMEM_SHARED((N, D_HID), jnp.float32),
            pltpu.VMEM_SHARED((N, DEG_W), jnp.float32),
            pltpu.VMEM((ROWS_BASE + 1, CHUNK), jnp.int32),
            pltpu.VMEM((ROWS_BASE + 1, CHUNK), jnp.int32),
            pltpu.VMEM((CHUNK, D_HID), jnp.float32),
            pltpu.VMEM((CHUNK, D_HID), jnp.float32),
            pltpu.VMEM((CHUNK, D_HID), jnp.float32),
            pltpu.VMEM((CHUNK, DEG_W), jnp.float32),
            pltpu.SemaphoreType.DMA,
            pltpu.SemaphoreType.DMA,
            pltpu.SemaphoreType.DMA,
            pltpu.SemaphoreType.DMA,
            pltpu.SemaphoreType.DMA,
            pltpu.SemaphoreType.DMA,
        ],
    )(_sc_agg_body)
    return f(h, edges3, z64, z16, ones)


def _fin_body(agg_ref, deg_ref, w_ref, b_ref, a_ref, o_ref):
    s = agg_ref[:, :D_HID] + agg_ref[:, D_HID:]
    d = deg_ref[:, 0:1] + deg_ref[:, DEG_W:DEG_W + 1]
    m = s / jnp.maximum(d, 1.0)
    p = jnp.where(m >= 0, m, a_ref[...] * m)
    o_ref[...] = (
        jnp.dot(p, w_ref[...], preferred_element_type=jnp.float32) + b_ref[...]
    )


def _stage3(aggc, degp, W2, b2, a):
    B = 1000
    a_row = jnp.full((1, D_HID), a, jnp.float32)
    return pl.pallas_call(
        _fin_body,
        grid=(N // B,),
        in_specs=[
            pl.BlockSpec((B, 2 * D_HID), lambda i: (i, 0)),
            pl.BlockSpec((B, 8 * DEG_W), lambda i: (i, 0)),
            pl.BlockSpec((D_HID, D_HID), lambda i: (0, 0)),
            pl.BlockSpec((1, D_HID), lambda i: (0, 0)),
            pl.BlockSpec((1, D_HID), lambda i: (0, 0)),
        ],
        out_specs=pl.BlockSpec((B, D_HID), lambda i: (i, 0)),
        out_shape=jax.ShapeDtypeStruct((N, D_HID), jnp.float32),
    )(aggc, degp, W2, b2.reshape(1, D_HID), a_row)


def kernel(x, edge_index, W1, b1, W2, b2, a):
    h = _stage1(x, W1, b1)
    aggc, degp = _stage2(h, edge_index)
    return _stage3(aggc, degp, W2, b2, a)


# speedup vs baseline: 15.7951x; 1.0271x over previous
import functools

import jax
import jax.numpy as jnp
from jax import lax
from jax.experimental import pallas as pl
from jax.experimental.pallas import tpu as pltpu
from jax.experimental.pallas import tpu_sc as plsc

N = 10000
E = 320000
D_IN = 128
D_HID = 64

NC = 2
NS = 16
NW = NC * NS
CHUNK = 128
EROWS = E // CHUNK
ROWS_BASE = EROWS // NW
ROWS_EXTRA = EROWS - ROWS_BASE * NW
NCHUNK3 = ROWS_BASE // 3
ROWS_PER_TILE = 624
ROWS_TAIL = N - ROWS_PER_TILE * NS
DEG_W = 16
RB = ROWS_PER_TILE // 16


def _mm1_body(x_ref, w_ref, b_ref, o_ref):
    o_ref[...] = (
        jnp.dot(x_ref[...], w_ref[...], preferred_element_type=jnp.float32)
        + b_ref[...]
    )


def _stage1(x, W1, b1):
    B = 1000
    x2 = x.reshape(N // 2, 2 * D_IN)
    wbd = jnp.zeros((2 * D_IN, 2 * D_HID), jnp.float32)
    wbd = wbd.at[:D_IN, :D_HID].set(W1).at[D_IN:, D_HID:].set(W1)
    bbd = jnp.concatenate([b1, b1]).reshape(1, 2 * D_HID)
    h2 = pl.pallas_call(
        _mm1_body,
        grid=(N // 2 // B,),
        in_specs=[
            pl.BlockSpec((B, 2 * D_IN), lambda i: (i, 0)),
            pl.BlockSpec((2 * D_IN, 2 * D_HID), lambda i: (0, 0)),
            pl.BlockSpec((1, 2 * D_HID), lambda i: (0, 0)),
        ],
        out_specs=pl.BlockSpec((B, 2 * D_HID), lambda i: (i, 0)),
        out_shape=jax.ShapeDtypeStruct((N // 2, 2 * D_HID), jnp.float32),
    )(x2, wbd, bbd)
    return h2.reshape(N, D_HID)


def _sc_agg_body(h_hbm, edges_hbm, z64_hbm, z16_hbm, ones_hbm,
                 agg_out, deg_out,
                 agg_sh, deg_sh, src_v, dst_v, rows0, rows1, rows2, ones_v,
                 sem_g0, sem_g1, sem_g2, sem_s0, sem_s1, sem_s2):
    cid = lax.axis_index("c")
    sid = lax.axis_index("s")
    wid = sid * NC + cid

    rbase = sid * ROWS_PER_TILE
    pltpu.sync_copy(z64_hbm.at[pl.ds(rbase, ROWS_PER_TILE)],
                    agg_sh.at[pl.ds(rbase, ROWS_PER_TILE)])
    pltpu.sync_copy(z16_hbm.at[pl.ds(rbase, ROWS_PER_TILE)],
                    deg_sh.at[pl.ds(rbase, ROWS_PER_TILE)])

    @pl.when(sid == NS - 1)
    def _zero_tail():
        tb = ROWS_PER_TILE * NS
        pltpu.sync_copy(z64_hbm.at[pl.ds(tb, ROWS_TAIL)],
                        agg_sh.at[pl.ds(tb, ROWS_TAIL)])
        pltpu.sync_copy(z16_hbm.at[pl.ds(tb, ROWS_TAIL)],
                        deg_sh.at[pl.ds(tb, ROWS_TAIL)])

    row_off = ROWS_BASE * wid + jnp.minimum(wid, ROWS_EXTRA)
    pltpu.sync_copy(edges_hbm.at[0, pl.ds(row_off, ROWS_BASE)],
                    src_v.at[pl.ds(0, ROWS_BASE)])
    pltpu.sync_copy(edges_hbm.at[1, pl.ds(row_off, ROWS_BASE)],
                    dst_v.at[pl.ds(0, ROWS_BASE)])

    @pl.when(wid < ROWS_EXTRA)
    def _stage_extra():
        pltpu.sync_copy(edges_hbm.at[0, pl.ds(row_off + ROWS_BASE, 1)],
                        src_v.at[pl.ds(ROWS_BASE, 1)])
        pltpu.sync_copy(edges_hbm.at[1, pl.ds(row_off + ROWS_BASE, 1)],
                        dst_v.at[pl.ds(ROWS_BASE, 1)])

    pltpu.sync_copy(ones_hbm, ones_v)
    plsc.subcore_barrier()

    def _fire_gather(j, buf, sem):
        pltpu.async_copy(h_hbm.at[src_v.at[j]], buf, sem)

    def _wait_gather(j, buf, sem):
        pltpu.make_async_copy(h_hbm.at[src_v.at[j]], buf, sem).wait()

    def _fire_scatter(j, buf, sem):
        pltpu.async_copy(buf, agg_sh.at[dst_v.at[j]], sem, add=True)
        pltpu.async_copy(ones_v, deg_sh.at[dst_v.at[j]], sem, add=True)

    def _wait_scatter(j, buf, sem):
        pltpu.make_async_copy(buf, agg_sh.at[dst_v.at[j]], sem).wait()
        pltpu.make_async_copy(ones_v, deg_sh.at[dst_v.at[j]], sem).wait()

    _fire_gather(0, rows0, sem_g0)

    def body(jj, carry):
        j0 = 3 * jj

        @pl.when(jj > 0)
        def _():
            _wait_scatter(j0 - 2, rows1, sem_s1)

        _fire_gather(j0 + 1, rows1, sem_g1)
        _wait_gather(j0, rows0, sem_g0)
        _fire_scatter(j0, rows0, sem_s0)

        @pl.when(jj > 0)
        def _():
            _wait_scatter(j0 - 1, rows2, sem_s2)

        _fire_gather(j0 + 2, rows2, sem_g2)
        _wait_gather(j0 + 1, rows1, sem_g1)
        _fire_scatter(j0 + 1, rows1, sem_s1)
        _wait_scatter(j0, rows0, sem_s0)

        @pl.when(jj + 1 < NCHUNK3)
        def _():
            _fire_gather(j0 + 3, rows0, sem_g0)

        _wait_gather(j0 + 2, rows2, sem_g2)
        _fire_scatter(j0 + 2, rows2, sem_s2)
        return carry

    lax.fori_loop(0, NCHUNK3, body, 0)
    _wait_scatter(ROWS_BASE - 2, rows1, sem_s1)
    _wait_scatter(ROWS_BASE - 1, rows2, sem_s2)

    @pl.when(wid < ROWS_EXTRA)
    def _extra_chunk():
        _fire_gather(ROWS_BASE, rows0, sem_g0)
        _wait_gather(ROWS_BASE, rows0, sem_g0)
        _fire_scatter(ROWS_BASE, rows0, sem_s0)
        _wait_scatter(ROWS_BASE, rows0, sem_s0)

    plsc.subcore_barrier()

    pltpu.sync_copy(agg_sh.at[pl.ds(rbase, ROWS_PER_TILE)],
                    agg_out.at[pl.ds(rbase, ROWS_PER_TILE),
                               pl.ds(cid * D_HID, D_HID)])

    pltpu.sync_copy(deg_sh.at[pl.ds(rbase, ROWS_PER_TILE)],
                    deg_out.at[pl.ds(rbase, ROWS_PER_TILE),
                               pl.ds(cid * DEG_W, DEG_W)])

    @pl.when(sid == NS - 1)
    def _publish_tail():
        tb = ROWS_PER_TILE * NS
        pltpu.sync_copy(agg_sh.at[pl.ds(tb, ROWS_TAIL)],
                        agg_out.at[pl.ds(tb, ROWS_TAIL),
                                   pl.ds(cid * D_HID, D_HID)])
        pltpu.sync_copy(deg_sh.at[pl.ds(tb, ROWS_TAIL)],
                        deg_out.at[pl.ds(tb, ROWS_TAIL),
                                   pl.ds(cid * DEG_W, DEG_W)])


def _stage2(h, edge_index):
    edges3 = edge_index.reshape(2, EROWS, CHUNK)
    z64 = jnp.zeros((N, D_HID), jnp.float32)
    z16 = jnp.zeros((N, DEG_W), jnp.float32)
    ones = jnp.ones((CHUNK, DEG_W), jnp.float32)
    mesh = plsc.VectorSubcoreMesh(core_axis_name="c", subcore_axis_name="s")
    f = functools.partial(
        pl.kernel,
        out_type=[
            jax.ShapeDtypeStruct((N, 2 * D_HID), jnp.float32),
            jax.ShapeDtypeStruct((N, 8 * DEG_W), jnp.float32),
        ],
        mesh=mesh,
        compiler_params=pltpu.CompilerParams(use_tc_tiling_on_sc=False),
        scratch_types=[
            pltpu.VMEM_SHARED((N, D_HID), jnp.float32),
            pltpu.VMEM_SHARED((N, DEG_W), jnp.float32),
            pltpu.VMEM((ROWS_BASE + 1, CHUNK), jnp.int32),
            pltpu.VMEM((ROWS_BASE + 1, CHUNK), jnp.int32),
            pltpu.VMEM((CHUNK, D_HID), jnp.float32),
            pltpu.VMEM((CHUNK, D_HID), jnp.float32),
            pltpu.VMEM((CHUNK, D_HID), jnp.float32),
            pltpu.VMEM((CHUNK, DEG_W), jnp.float32),
            pltpu.SemaphoreType.DMA,
            pltpu.SemaphoreType.DMA,
            pltpu.SemaphoreType.DMA,
            pltpu.SemaphoreType.DMA,
            pltpu.SemaphoreType.DMA,
            pltpu.SemaphoreType.DMA,
        ],
    )(_sc_agg_body)
    return f(h, edges3, z64, z16, ones)


def _fin_body(agg_ref, deg_ref, w_ref, b_ref, a_ref, o_ref):
    s = agg_ref[:, :D_HID] + agg_ref[:, D_HID:]
    d = deg_ref[:, 0:1] + deg_ref[:, DEG_W:DEG_W + 1]
    m = s / jnp.maximum(d, 1.0)
    p = jnp.where(m >= 0, m, a_ref[...] * m)
    o_ref[...] = (
        jnp.dot(p, w_ref[...], preferred_element_type=jnp.float32) + b_ref[...]
    )


def _stage3(aggc, degp, W2, b2, a):
    B = 2000
    a_row = jnp.full((1, D_HID), a, jnp.float32)
    return pl.pallas_call(
        _fin_body,
        grid=(N // B,),
        in_specs=[
            pl.BlockSpec((B, 2 * D_HID), lambda i: (i, 0)),
            pl.BlockSpec((B, 8 * DEG_W), lambda i: (i, 0)),
            pl.BlockSpec((D_HID, D_HID), lambda i: (0, 0)),
            pl.BlockSpec((1, D_HID), lambda i: (0, 0)),
            pl.BlockSpec((1, D_HID), lambda i: (0, 0)),
        ],
        out_specs=pl.BlockSpec((B, D_HID), lambda i: (i, 0)),
        out_shape=jax.ShapeDtypeStruct((N, D_HID), jnp.float32),
    )(aggc, degp, W2, b2.reshape(1, D_HID), a_row)


def kernel(x, edge_index, W1, b1, W2, b2, a):
    h = _stage1(x, W1, b1)
    aggc, degp = _stage2(h, edge_index)
    return _stage3(aggc, degp, W2, b2, a)
